# baseline (speedup 1.0000x reference)
import jax
import jax.numpy as jnp
from jax import lax
from jax.experimental import pallas as pl
from jax.experimental.pallas import tpu as pltpu
from jax.experimental.pallas import tpu_sc as plsc

B = 1024
M = 100000
L = 16384

NC = 2
NS = 16
NW = NC * NS

CH = 32
NCHUNK = M // CH
SB = 16
NSTARTS = NCHUNK + 1


def _scatter_body(tab_hbm, x0t_hbm, inds_hbm, order_hbm, starts_hbm, res_hbm,
                  cbuf, stage, inds_v, order_v, starts_v, jb, sin, sout, sg):
    wid = lax.axis_index("s") * NC + lax.axis_index("c")
    c0 = (wid * NCHUNK) // NW
    c1 = ((wid + 1) * NCHUNK) // NW
    nck = c1 - c0
    lane = lax.iota(jnp.int32, 16)

    pltpu.sync_copy(inds_hbm, inds_v.at[pl.ds(0, L)])
    pltpu.sync_copy(order_hbm, order_v.at[pl.ds(0, L)])
    pltpu.sync_copy(starts_hbm, starts_v.at[pl.ds(0, NSTARTS)])

    def vext(ref, idx):
        blk = ref[pl.ds((idx >> 4) << 4, 16)]
        return jnp.sum(jnp.where(lane == (idx & 15), blk, 0))

    def load(k, s):
        r0 = (c0 + k) * CH
        return pltpu.make_async_copy(
            tab_hbm.at[pl.ds(r0, CH)], cbuf.at[s], sin.at[s])

    def store(k, s):
        r0 = (c0 + k) * CH
        return pltpu.make_async_copy(
            cbuf.at[s], res_hbm.at[pl.ds(r0, CH)], sout.at[s])

    load(0, 0).start()

    def chunk_loop(k, carry):
        s = k & 1
        cg = c0 + k

        @pl.when(k + 1 < nck)
        def _():
            @pl.when(k >= 1)
            def _():
                store(k - 1, 1 - s).wait()
            load(k + 1, 1 - s).start()

        e0 = vext(starts_v, cg)
        e1 = vext(starts_v, cg + 1)
        n_c = e1 - e0
        nb = (n_c + SB - 1) // SB

        jb[...] = order_v[pl.ds(e0, 16)] & (L - 1)
        gather = pltpu.make_async_copy(x0t_hbm.at[jb], stage, sg)

        @pl.when(n_c > 0)
        def _():
            gather.start()

        load(k, s).wait()

        def batch_loop(b, carry2):
            gather.wait()
            rows_v = inds_v[pl.ds(e0 + b * 16, 16)] - cg * CH
            for r in range(SB):
                @pl.when(b * 16 + r < n_c)
                def _(r=r, s=s, b=b):
                    lrow = jnp.sum(jnp.where(lane == r, rows_v, 0))

                    def col_add(q, c3, r=r, lrow=lrow, s=s):
                        base = q * 128
                        for u in range(8):
                            plsc.addupdate(
                                cbuf.at[s, lrow, pl.ds(base + u * 16, 16)],
                                stage[r, pl.ds(base + u * 16, 16)])
                        return c3

                    lax.fori_loop(0, B // 128, col_add, 0)

            @pl.when(b + 1 < nb)
            def _(b=b):
                jb[...] = order_v[pl.ds(e0 + (b + 1) * 16, 16)] & (L - 1)
                gather.start()

            return carry2

        lax.fori_loop(0, nb, batch_loop, 0)

        store(k, s).start()
        return carry

    lax.fori_loop(0, nck, chunk_loop, 0)

    @pl.when(nck >= 2)
    def _():
        store(nck - 2, nck & 1).wait()

    store(nck - 1, (nck - 1) & 1).wait()


def kernel(out, x0, ind1):
    tabT = out.T
    x0T = jnp.transpose(x0)
    order = jnp.argsort(ind1).astype(jnp.int32)
    ind_sorted = jnp.take(ind1, order)
    starts = jnp.searchsorted(
        ind_sorted, jnp.arange(0, M + 1, CH, dtype=jnp.int32)
    ).astype(jnp.int32)
    mesh = plsc.VectorSubcoreMesh(core_axis_name="c", subcore_axis_name="s")
    k = pl.kernel(
        _scatter_body,
        out_type=jax.ShapeDtypeStruct((M, B), jnp.float32),
        mesh=mesh,
        scratch_types=[
            pltpu.VMEM((2, CH, B), jnp.float32),
            pltpu.VMEM((SB, B), jnp.float32),
            pltpu.VMEM((L + 16,), jnp.int32),
            pltpu.VMEM((L + 16,), jnp.int32),
            pltpu.VMEM((NSTARTS + 15, ), jnp.int32),
            pltpu.VMEM((16,), jnp.int32),
            pltpu.SemaphoreType.DMA((2,)),
            pltpu.SemaphoreType.DMA((2,)),
            pltpu.SemaphoreType.DMA,
        ],
        compiler_params=pltpu.CompilerParams(needs_layout_passes=False),
    )
    resT = k(tabT, x0T, ind_sorted, order, starts)
    return resT.T


# speedup vs baseline: 1.0092x; 1.0092x over previous
import jax
import jax.numpy as jnp
from jax import lax
from jax.experimental import pallas as pl
from jax.experimental.pallas import tpu as pltpu
from jax.experimental.pallas import tpu_sc as plsc

B = 1024
M = 100000
L = 16384

NC = 2
NS = 16
NW = NC * NS

CH = 32
NCHUNK = M // CH
SB = 16
NSTARTS = NCHUNK + 1


def _scatter_body(tab_hbm, x0t_hbm, packed_hbm, starts_hbm, res_hbm,
                  cbuf, stage, packed_v, starts_v, jb, sin, sout, sg):
    wid = lax.axis_index("s") * NC + lax.axis_index("c")
    c0 = (wid * NCHUNK) // NW
    c1 = ((wid + 1) * NCHUNK) // NW
    nck = c1 - c0
    lane = lax.iota(jnp.int32, 16)

    pltpu.sync_copy(packed_hbm, packed_v.at[pl.ds(0, L)])
    pltpu.sync_copy(starts_hbm, starts_v.at[pl.ds(0, NSTARTS)])

    def vext(ref, idx):
        blk = ref[pl.ds((idx >> 4) << 4, 16)]
        return jnp.sum(jnp.where(lane == (idx & 15), blk, 0))

    def load(k, s):
        r0 = (c0 + k) * CH
        return pltpu.make_async_copy(
            tab_hbm.at[pl.ds(r0, CH)], cbuf.at[s], sin.at[s])

    def store(k, s):
        r0 = (c0 + k) * CH
        return pltpu.make_async_copy(
            cbuf.at[s], res_hbm.at[pl.ds(r0, CH)], sout.at[s])

    load(0, 0).start()

    def chunk_loop(k, carry):
        s = k & 1
        cg = c0 + k

        @pl.when(k + 1 < nck)
        def _():
            @pl.when(k >= 1)
            def _():
                store(k - 1, 1 - s).wait()
            load(k + 1, 1 - s).start()

        e0 = vext(starts_v, cg)
        e1 = vext(starts_v, cg + 1)
        n_c = e1 - e0
        nb = (n_c + SB - 1) // SB

        jb[...] = packed_v[pl.ds(e0, 16)] & (L - 1)
        gather = pltpu.make_async_copy(x0t_hbm.at[jb], stage, sg)

        @pl.when(n_c > 0)
        def _():
            gather.start()

        load(k, s).wait()

        def batch_loop(b, carry2):
            gather.wait()
            rows_v = (packed_v[pl.ds(e0 + b * 16, 16)] >> 14) - cg * CH
            for r in range(SB):
                @pl.when(b * 16 + r < n_c)
                def _(r=r, s=s, b=b):
                    lrow = jnp.sum(jnp.where(lane == r, rows_v, 0))

                    def col_add(q, c3, r=r, lrow=lrow, s=s):
                        base = q * 128
                        for u in range(8):
                            plsc.addupdate(
                                cbuf.at[s, lrow, pl.ds(base + u * 16, 16)],
                                stage[r, pl.ds(base + u * 16, 16)])
                        return c3

                    lax.fori_loop(0, B // 128, col_add, 0)

            @pl.when(b + 1 < nb)
            def _(b=b):
                jb[...] = packed_v[pl.ds(e0 + (b + 1) * 16, 16)] & (L - 1)
                gather.start()

            return carry2

        lax.fori_loop(0, nb, batch_loop, 0)

        store(k, s).start()
        return carry

    lax.fori_loop(0, nck, chunk_loop, 0)

    @pl.when(nck >= 2)
    def _():
        store(nck - 2, nck & 1).wait()

    store(nck - 1, (nck - 1) & 1).wait()


def kernel(out, x0, ind1):
    tabT = out.T
    x0T = jnp.transpose(x0)
    packed = jnp.sort(
        (ind1 << 14) | jnp.arange(L, dtype=jnp.int32))
    starts = jnp.searchsorted(
        packed, jnp.arange(0, M + 1, CH, dtype=jnp.int32) << 14
    ).astype(jnp.int32)
    mesh = plsc.VectorSubcoreMesh(core_axis_name="c", subcore_axis_name="s")
    k = pl.kernel(
        _scatter_body,
        out_type=jax.ShapeDtypeStruct((M, B), jnp.float32),
        mesh=mesh,
        scratch_types=[
            pltpu.VMEM((2, CH, B), jnp.float32),
            pltpu.VMEM((SB, B), jnp.float32),
            pltpu.VMEM((L + 16,), jnp.int32),
            pltpu.VMEM((NSTARTS + 15, ), jnp.int32),
            pltpu.VMEM((16,), jnp.int32),
            pltpu.SemaphoreType.DMA((2,)),
            pltpu.SemaphoreType.DMA((2,)),
            pltpu.SemaphoreType.DMA,
        ],
        compiler_params=pltpu.CompilerParams(needs_layout_passes=False),
    )
    resT = k(tabT, x0T, packed, starts)
    return resT.T
